# baseline (speedup 1.0000x reference)
import functools

import jax
import jax.numpy as jnp
from jax import lax
from jax.experimental import pallas as pl
from jax.experimental.pallas import tpu as pltpu
from jax.experimental.pallas import tpu_sc as plsc

N = 10000
E = 320000
D = 128
ED = 16
H = 8
O = 64
HO = 512
B = 256

NC = 2
NS = 16
NW = NC * NS
L = 16

NEG = -3.0e38

_MESH = plsc.VectorSubcoreMesh(core_axis_name="c", subcore_axis_name="s")
_SC_PARAMS = pltpu.CompilerParams(use_tc_tiling_on_sc=False)


def _mm2_body(x_ref, wl_ref, bl_ref, wr_ref, br_ref, ol_ref, or_ref):
    xb = x_ref[...]
    ol_ref[...] = jnp.dot(xb, wl_ref[...], preferred_element_type=jnp.float32) + bl_ref[...]
    or_ref[...] = jnp.dot(xb, wr_ref[...], preferred_element_type=jnp.float32) + br_ref[...]


def _node_mm0(x, Wl, bl, Wr, br):
    blk = 1000
    grid = N // blk
    return pl.pallas_call(
        _mm2_body,
        grid=(grid,),
        in_specs=[
            pl.BlockSpec((blk, D), lambda i: (i, 0)),
            pl.BlockSpec((D, HO), lambda i: (0, 0)),
            pl.BlockSpec((1, HO), lambda i: (0, 0)),
            pl.BlockSpec((D, HO), lambda i: (0, 0)),
            pl.BlockSpec((1, HO), lambda i: (0, 0)),
        ],
        out_specs=[
            pl.BlockSpec((blk, HO), lambda i: (i, 0)),
            pl.BlockSpec((blk, HO), lambda i: (i, 0)),
        ],
        out_shape=[
            jax.ShapeDtypeStruct((N, HO), jnp.float32),
            jax.ShapeDtypeStruct((N, HO), jnp.float32),
        ],
    )(x, Wl, bl.reshape(1, HO), Wr, br.reshape(1, HO))


def _mm2h_body(o0_ref, b0_ref, wl_ref, bl_ref, wr_ref, br_ref, ol_ref, or_ref):
    accl = jnp.zeros(ol_ref.shape, jnp.float32)
    accr = jnp.zeros(or_ref.shape, jnp.float32)
    for h in range(H):
        p, k = h // 2, h % 2
        hb = o0_ref[p][:, k * O:(k + 1) * O] + b0_ref[0, h]
        accl = accl + jnp.dot(hb, wl_ref[h], preferred_element_type=jnp.float32)
        accr = accr + jnp.dot(hb, wr_ref[h], preferred_element_type=jnp.float32)
    ol_ref[...] = accl + bl_ref[...]
    or_ref[...] = accr + br_ref[...]


def _node_mm1(o0, bias0, Wl, bl, Wr, br):
    blk = 1024
    grid = N2 // blk
    return pl.pallas_call(
        _mm2h_body,
        grid=(grid,),
        in_specs=[
            pl.BlockSpec((H // 2, blk, 2 * O), lambda i: (0, i, 0)),
            pl.BlockSpec((1, H, O), lambda i: (0, 0, 0)),
            pl.BlockSpec((H, O, HO), lambda i: (0, 0, 0)),
            pl.BlockSpec((1, HO), lambda i: (0, 0)),
            pl.BlockSpec((H, O, HO), lambda i: (0, 0, 0)),
            pl.BlockSpec((1, HO), lambda i: (0, 0)),
        ],
        out_specs=[
            pl.BlockSpec((blk, HO), lambda i: (i, 0)),
            pl.BlockSpec((blk, HO), lambda i: (i, 0)),
        ],
        out_shape=[
            jax.ShapeDtypeStruct((N2, HO), jnp.float32),
            jax.ShapeDtypeStruct((N2, HO), jnp.float32),
        ],
    )(o0, bias0.reshape(1, H, O), Wl.reshape(H, O, HO), bl.reshape(1, HO),
      Wr.reshape(H, O, HO), br.reshape(1, HO))


def _emm_body(ea_ref, we_ref, out_ref):
    out_ref[...] = jnp.dot(ea_ref[...], we_ref[...], preferred_element_type=jnp.float32)


def _edge_mm(edge_attr, We):
    blk = 2000
    grid = E // blk
    return pl.pallas_call(
        _emm_body,
        grid=(grid,),
        in_specs=[
            pl.BlockSpec((blk, ED), lambda i: (i, 0)),
            pl.BlockSpec((ED, HO), lambda i: (0, 0)),
        ],
        out_specs=pl.BlockSpec((blk, HO), lambda i: (i, 0)),
        out_shape=jax.ShapeDtypeStruct((E, HO), jnp.float32),
    )(edge_attr, We)


N2 = 10240
NROW2 = N2 // NS
CA = 40
EPW_A = E // NW


def _passA_body(xl_hbm, xr_hbm, e_hbm, src_hbm, dst_hbm, att_hbm,
                alpha_hbm, mx_hbm,
                sidx_v, didx_v, gl_v, gr_v, ev_v, att_v, al_v, mx_v, sem):
    c = lax.axis_index("c")
    s = lax.axis_index("s")
    wid = s * NC + c
    base0 = wid * EPW_A
    pltpu.sync_copy(att_hbm, att_v)
    att_regs = [att_v[h, pl.ds(j * L, L)] for h in range(H) for j in range(O // L)]
    iota16 = lax.iota(jnp.int32, L)
    p8, p4, p2, p1 = (iota16 ^ st for st in (8, 4, 2, 1))
    m8 = iota16 < 8
    m4 = (iota16 & 4) == 0
    m2 = (iota16 & 2) == 0
    pfin = jnp.where(iota16 < 8,
                     ((iota16 & 1) << 3) | (iota16 & 2) << 1 | ((iota16 & 4) >> 1),
                     0)

    def chunk_body(k, mx_carry):
        base = base0 + k * CA
        pltpu.sync_copy(src_hbm.at[pl.ds(base, CA)], sidx_v)
        pltpu.sync_copy(dst_hbm.at[pl.ds(base, CA)], didx_v)
        pltpu.sync_copy(e_hbm.at[pl.ds(base, CA)], ev_v)
        pltpu.async_copy(xl_hbm.at[sidx_v], gl_v, sem).wait()
        pltpu.async_copy(xr_hbm.at[didx_v], gr_v, sem).wait()

        def edge_body(i, mxv):
            accs = []
            for h in range(H):
                acc = jnp.zeros((L,), jnp.float32)
                for j in range(O // L):
                    sl = pl.ds(h * O + j * L, L)
                    z = gl_v[i, sl] + gr_v[i, sl] + ev_v[i, sl]
                    m = jnp.maximum(z, jnp.float32(0.2) * z)
                    acc = acc + m * att_regs[h * (O // L) + j]
                accs.append(acc)
            t = [a + jnp.take(a, p8, axis=0) for a in accs]
            u = [jnp.where(m8, t[2 * p], jnp.take(t[2 * p + 1], p8, axis=0))
                 for p in range(4)]
            v = [a + jnp.take(a, p4, axis=0) for a in u]
            w = [jnp.where(m4, v[2 * q], jnp.take(v[2 * q + 1], p4, axis=0))
                 for q in range(2)]
            x = [a + jnp.take(a, p2, axis=0) for a in w]
            y = jnp.where(m2, x[0], jnp.take(x[1], p2, axis=0))
            zv = y + jnp.take(y, p1, axis=0)
            row = jnp.take(zv, pfin, axis=0)
            al_v[i, :] = row
            return jnp.maximum(mxv, row)

        mxv2 = lax.fori_loop(0, CA, edge_body, mx_carry, unroll=False)
        pltpu.sync_copy(al_v, alpha_hbm.at[pl.ds(base, CA)])
        return mxv2

    mx_fin = lax.fori_loop(0, EPW_A // CA, chunk_body,
                           jnp.full((L,), jnp.float32(NEG)), unroll=False)
    mx_v[...] = mx_fin
    pltpu.sync_copy(mx_v, mx_hbm.at[pl.ds(wid * L, L)])


def _passA(xl, xr, e, src, dst, att):
    return pl.kernel(
        _passA_body,
        out_type=[
            jax.ShapeDtypeStruct((E, L), jnp.float32),
            jax.ShapeDtypeStruct((NW * L,), jnp.float32),
        ],
        mesh=_MESH,
        compiler_params=_SC_PARAMS,
        scratch_types=[
            pltpu.VMEM((CA,), jnp.int32),
            pltpu.VMEM((CA,), jnp.int32),
            pltpu.VMEM((CA, HO), jnp.float32),
            pltpu.VMEM((CA, HO), jnp.float32),
            pltpu.VMEM((CA, HO), jnp.float32),
            pltpu.VMEM((H, O), jnp.float32),
            pltpu.VMEM((CA, L), jnp.float32),
            pltpu.VMEM((L,), jnp.float32),
            pltpu.SemaphoreType.DMA,
        ],
    )(xl, xr, e, src, dst, att)


CB1 = 80
EPW_B1 = E // NW


def _passB1_body(alpha_hbm, mx_hbm, dst_hbm,
                 den_hbm,
                 albuf_v, didx_v, mxall_v, stage_v, den_sh, sem):
    c = lax.axis_index("c")
    s = lax.axis_index("s")
    wid = s * NC + c
    zero = jnp.zeros((L,), jnp.float32)

    def zrow(r, cc):
        stage_v[r, :] = zero
        return cc

    lax.fori_loop(0, CB1, zrow, 0, unroll=False)
    for u in range(NROW2 // CB1):
        pltpu.sync_copy(stage_v, den_sh.at[pl.ds(s * NROW2 + u * CB1, CB1)])
    plsc.subcore_barrier()

    pltpu.sync_copy(mx_hbm, mxall_v)
    mv = jnp.full((L,), jnp.float32(NEG))
    for w in range(NW):
        mv = jnp.maximum(mv, mxall_v[pl.ds(w * L, L)])

    def chunk_body(k, carry):
        base = wid * EPW_B1 + k * CB1
        pltpu.sync_copy(alpha_hbm.at[pl.ds(base, CB1)], albuf_v)
        pltpu.sync_copy(dst_hbm.at[pl.ds(base, CB1)], didx_v)

        def edge_body(r, cc):
            albuf_v[r, :] = jnp.exp(albuf_v[r, :] - mv)
            return cc

        lax.fori_loop(0, CB1, edge_body, 0, unroll=False)
        pltpu.sync_copy(albuf_v, den_sh.at[didx_v], add=True)
        return carry

    lax.fori_loop(0, EPW_B1 // CB1, chunk_body, 0, unroll=False)
    plsc.subcore_barrier()
    for u in range(NROW2 // CB1):
        pltpu.sync_copy(den_sh.at[pl.ds(s * NROW2 + u * CB1, CB1)], stage_v)
        pltpu.sync_copy(stage_v,
                        den_hbm.at[c, pl.ds(s * NROW2 + u * CB1, CB1)])


def _passB1(alpha, mx, dst):
    return pl.kernel(
        _passB1_body,
        out_type=jax.ShapeDtypeStruct((NC, N2, L), jnp.float32),
        mesh=_MESH,
        compiler_params=_SC_PARAMS,
        scratch_types=[
            pltpu.VMEM((CB1, L), jnp.float32),
            pltpu.VMEM((CB1,), jnp.int32),
            pltpu.VMEM((NW * L,), jnp.float32),
            pltpu.VMEM((CB1, L), jnp.float32),
            pltpu.VMEM_SHARED((N2, L), jnp.float32),
            pltpu.SemaphoreType.DMA,
        ],
    )(alpha, mx, dst)


def _densum_body(dp_ref, out_ref):
    out_ref[...] = dp_ref[0] + dp_ref[1]


def _densum(dp):
    blk = 2048
    return pl.pallas_call(
        _densum_body,
        grid=(N2 // blk,),
        in_specs=[pl.BlockSpec((NC, blk, L), lambda i: (0, i, 0))],
        out_specs=pl.BlockSpec((blk, L), lambda i: (i, 0)),
        out_shape=jax.ShapeDtypeStruct((N2, L), jnp.float32),
    )(dp)


CB2 = 80
EPW_B2 = E // NW


def _passB2_body(alpha_hbm, mx_hbm, den_hbm, dst_hbm,
                 a_hbm,
                 exbuf_v, denb_v, didx_v, mxall_v, sem):
    c = lax.axis_index("c")
    s = lax.axis_index("s")
    wid = s * NC + c
    eps = jnp.float32(1e-16)

    pltpu.sync_copy(mx_hbm, mxall_v)
    mv = jnp.full((L,), jnp.float32(NEG))
    for w in range(NW):
        mv = jnp.maximum(mv, mxall_v[pl.ds(w * L, L)])

    def chunk_body(k, carry):
        base = wid * EPW_B2 + k * CB2
        pltpu.sync_copy(alpha_hbm.at[pl.ds(base, CB2)], exbuf_v)
        pltpu.sync_copy(dst_hbm.at[pl.ds(base, CB2)], didx_v)
        pltpu.async_copy(den_hbm.at[didx_v], denb_v, sem).wait()

        def edge_body(r, cc):
            exbuf_v[r, :] = jnp.exp(exbuf_v[r, :] - mv) / (denb_v[r, :] + eps)
            return cc

        lax.fori_loop(0, CB2, edge_body, 0, unroll=False)
        pltpu.sync_copy(exbuf_v, a_hbm.at[pl.ds(base, CB2)])
        return carry

    lax.fori_loop(0, EPW_B2 // CB2, chunk_body, 0, unroll=False)


def _passB2(alpha, mx, den, dst):
    return pl.kernel(
        _passB2_body,
        out_type=jax.ShapeDtypeStruct((E, L), jnp.float32),
        mesh=_MESH,
        compiler_params=_SC_PARAMS,
        scratch_types=[
            pltpu.VMEM((CB2, L), jnp.float32),
            pltpu.VMEM((CB2, L), jnp.float32),
            pltpu.VMEM((CB2,), jnp.int32),
            pltpu.VMEM((NW * L,), jnp.float32),
            pltpu.SemaphoreType.DMA,
        ],
    )(alpha, mx, den, dst)


CC = 160
SW = 80
EPW_C = E // NS
HPC = H // NC
PPC = HPC // 2
W2 = 2 * O


def _passC_body(concat, a_hbm, src2_hbm, dst2_hbm, xlr_hbm,
                out_hbm,
                sidx_v, didx_v, gidx_v, a_v, g_v, acc_sh, sem):
    c = lax.axis_index("c")
    s = lax.axis_index("s")
    zero = jnp.zeros((L,), jnp.float32)
    stage_v = g_v.at[pl.ds(0, SW)]

    def zero_acc():
        def zrow(r, cc):
            for j in range(W2 // L):
                g_v[r, pl.ds(j * L, L)] = zero
            return cc

        lax.fori_loop(0, SW, zrow, 0, unroll=False)
        for u in range(NROW2 // SW):
            pltpu.sync_copy(stage_v, acc_sh.at[pl.ds(s * NROW2 + u * SW, SW)])

    def dump_acc(dst_slot):
        for u in range(NROW2 // SW):
            pltpu.sync_copy(acc_sh.at[pl.ds(s * NROW2 + u * SW, SW)], stage_v)
            pltpu.sync_copy(stage_v,
                            out_hbm.at[dst_slot, pl.ds(s * NROW2 + u * SW, SW)])

    for p_local in range(PPC):
        pair_abs = c * PPC + p_local
        h0 = 2 * pair_abs
        if concat or p_local == 0:
            zero_acc()
            plsc.subcore_barrier()

        def chunk_body(k, carry):
            base = s * EPW_C + k * CC
            pltpu.sync_copy(src2_hbm.at[pl.ds(base // SW, CC // SW)], sidx_v)
            pltpu.sync_copy(dst2_hbm.at[pl.ds(base // SW, CC // SW)], didx_v)
            pltpu.sync_copy(a_hbm.at[pl.ds(base, CC)], a_v)

            def idx_body(t, cc):
                for q in range(SW // L):
                    sl = pl.ds(q * L, L)
                    gidx_v[t, sl] = sidx_v[t, sl] * (H // 2) + pair_abs
                return cc

            lax.fori_loop(0, CC // SW, idx_body, 0, unroll=False)
            descs = [pltpu.async_copy(xlr_hbm.at[gidx_v.at[u]],
                                      g_v.at[pl.ds(u * SW, SW)], sem)
                     for u in range(CC // SW)]
            for d in descs:
                d.wait()

            def edge_body(r, cc):
                av0 = jnp.take(a_v[r, :], jnp.full((L,), h0, jnp.int32),
                               axis=0)
                av1 = jnp.take(a_v[r, :], jnp.full((L,), h0 + 1, jnp.int32),
                               axis=0)
                for j in range(O // L):
                    sl = pl.ds(j * L, L)
                    g_v[r, sl] = g_v[r, sl] * av0
                for j in range(O // L):
                    sl = pl.ds(O + j * L, L)
                    g_v[r, sl] = g_v[r, sl] * av1
                return cc

            lax.fori_loop(0, CC, edge_body, 0, unroll=False)
            for u in range(CC // SW):
                pltpu.sync_copy(g_v.at[pl.ds(u * SW, SW)],
                                acc_sh.at[didx_v.at[u]], add=True)
            return carry

        lax.fori_loop(0, EPW_C // CC, chunk_body, 0, unroll=False)
        plsc.subcore_barrier()
        if concat:
            dump_acc(pair_abs)
            plsc.subcore_barrier()
    if not concat:
        dump_acc(c)


def _passC(a, src2, dst2, xlr, concat):
    slots = H // 2 if concat else NC
    return pl.kernel(
        functools.partial(_passC_body, concat),
        out_type=jax.ShapeDtypeStruct((slots, N2, W2), jnp.float32),
        mesh=_MESH,
        compiler_params=_SC_PARAMS,
        scratch_types=[
            pltpu.VMEM((CC // SW, SW), jnp.int32),
            pltpu.VMEM((CC // SW, SW), jnp.int32),
            pltpu.VMEM((CC // SW, SW), jnp.int32),
            pltpu.VMEM((CC, L), jnp.float32),
            pltpu.VMEM((CC, W2), jnp.float32),
            pltpu.VMEM_SHARED((N2, W2), jnp.float32),
            pltpu.SemaphoreType.DMA,
        ],
    )(a, src2, dst2, xlr)


PPB = 1024


def _prep_body(p0_ref, p1_ref, bias1_ref, wa_ref, ba_ref, h_ref, w_ref):
    ps = p0_ref[...] + p1_ref[...]
    h = (ps[:, :O] + ps[:, O:]) * jnp.float32(1.0 / H) + bias1_ref[...]
    z = jnp.dot(h, wa_ref[...], preferred_element_type=jnp.float32) + ba_ref[...]
    h_ref[...] = h
    w_ref[...] = jnp.float32(1.0) / (jnp.float32(1.0) + jnp.exp(-z))


def _prep(p0, p1, bias1, Wa, ba):
    grid = N2 // PPB
    return pl.pallas_call(
        _prep_body,
        grid=(grid,),
        in_specs=[
            pl.BlockSpec((PPB, W2), lambda i: (i, 0)),
            pl.BlockSpec((PPB, W2), lambda i: (i, 0)),
            pl.BlockSpec((1, O), lambda i: (0, 0)),
            pl.BlockSpec((O, 1), lambda i: (0, 0)),
            pl.BlockSpec((1, 1), lambda i: (0, 0)),
        ],
        out_specs=[
            pl.BlockSpec((PPB, O), lambda i: (i, 0)),
            pl.BlockSpec((PPB, 1), lambda i: (i, 0)),
        ],
        out_shape=[
            jax.ShapeDtypeStruct((N2, O), jnp.float32),
            jax.ShapeDtypeStruct((N2, 1), jnp.float32),
        ],
    )(p0, p1, bias1.reshape(1, O), Wa, ba.reshape(1, 1))


RPW = N2 // NW
BT = 264


def _scpool_body(h_hbm, w_hbm, batch_hbm, out_hbm,
                 h_v, w_v, b_v, tab1_v, tab2_v, sem):
    c = lax.axis_index("c")
    s = lax.axis_index("s")
    wid = s * NC + c
    rows0 = wid * RPW
    pltpu.sync_copy(h_hbm.at[pl.ds(rows0, RPW)], h_v)
    pltpu.sync_copy(w_hbm.at[pl.ds(rows0, RPW)], w_v)
    pltpu.sync_copy(batch_hbm.at[pl.ds(rows0, RPW)], b_v)
    zero = jnp.zeros((L,), jnp.float32)
    neg = jnp.full((L,), jnp.float32(NEG))

    def zrow(r, cc):
        for j in range(O // L):
            tab1_v[r, pl.ds(j * L, L)] = neg
            tab2_v[r, pl.ds(j * L, L)] = zero
        return cc

    lax.fori_loop(0, BT, zrow, 0, unroll=False)

    def grp(g, cc):
        seg16 = b_v[pl.ds(g * L, L)]
        w16 = w_v[pl.ds(g * L, L)]
        for t in range(L):
            r = g * L + t
            seg = seg16[t]
            wb = jnp.full((L,), w16[t])
            for j in range(O // L):
                sl = pl.ds(j * L, L)
                hv = h_v[r, sl]
                tab1_v[seg, sl] = jnp.maximum(tab1_v[seg, sl], hv)
                tab2_v[seg, sl] = tab2_v[seg, sl] + hv * wb
        return cc

    lax.fori_loop(0, RPW // L, grp, 0, unroll=False)
    pltpu.sync_copy(tab1_v.at[pl.ds(0, B)], out_hbm.at[0, wid])
    pltpu.sync_copy(tab2_v.at[pl.ds(0, B)], out_hbm.at[1, wid])


def _scpool(h, w, batch_pad):
    return pl.kernel(
        _scpool_body,
        out_type=jax.ShapeDtypeStruct((2, NW, B, O), jnp.float32),
        mesh=_MESH,
        compiler_params=_SC_PARAMS,
        scratch_types=[
            pltpu.VMEM((RPW, O), jnp.float32),
            pltpu.VMEM((RPW,), jnp.float32),
            pltpu.VMEM((RPW,), jnp.int32),
            pltpu.VMEM((BT, O), jnp.float32),
            pltpu.VMEM((BT, O), jnp.float32),
            pltpu.SemaphoreType.DMA,
        ],
    )(h, w, batch_pad)


def _mlp_body(g1p_ref, g2p_ref, w1_ref, b1_ref, ap_ref, w2_ref, b2_ref, out_ref):
    g1 = jnp.max(g1p_ref[...], axis=0)
    g1 = jnp.where(g1 <= jnp.float32(-1e38), jnp.float32(0.0), g1)
    g2 = jnp.sum(g2p_ref[...], axis=0)
    g = jnp.concatenate([g1, g2], axis=1)
    gm = jnp.dot(g, w1_ref[...], preferred_element_type=jnp.float32) + b1_ref[...]
    gm = jnp.where(gm >= 0, gm, ap_ref[0, 0] * gm)
    out_ref[...] = jnp.dot(gm, w2_ref[...], preferred_element_type=jnp.float32) + b2_ref[...]


def _mlp(g1p, g2p, W1, b1, a_prelu, W2, b2):
    return pl.pallas_call(
        _mlp_body,
        out_shape=jax.ShapeDtypeStruct((B, 1), jnp.float32),
    )(g1p, g2p, W1, b1.reshape(1, -1), a_prelu.reshape(1, 1), W2, b2.reshape(1, 1))


def _gat_layer_sc(xl, xr, e, src, dst, src2, dst2, att, concat):
    alpha, mx = _passA(xl, xr, e, src, dst, att)
    denp = _passB1(alpha, mx, dst)
    a = _passB2(alpha, mx, _densum(denp), dst)
    nn = xl.shape[0]
    return _passC(a, src2, dst2, xl.reshape(nn * (H // 2), W2), concat)


def kernel(x, edge_index, edge_attr, batch, Wl0, bl0, Wr0, br0, We0, att0,
           bias0, Wl1, bl1, Wr1, br1, We1, att1, bias1, Wa, ba, W1, b1,
           a_prelu, W2, b2):
    src = edge_index[0]
    dst = edge_index[1]
    src2 = src.reshape(E // 80, 80)
    dst2 = dst.reshape(E // 80, 80)
    batch_pad = jnp.concatenate(
        [batch, jnp.full((N2 - N,), B, jnp.int32)])

    xl0, xr0 = _node_mm0(x, Wl0, bl0, Wr0, br0)
    e0 = _edge_mm(edge_attr, We0)
    out0 = _gat_layer_sc(xl0, xr0, e0, src, dst, src2, dst2, att0, True)

    xl1, xr1 = _node_mm1(out0, bias0, Wl1, bl1, Wr1, br1)
    e1 = _edge_mm(edge_attr, We1)
    out1p = _gat_layer_sc(xl1, xr1, e1, src, dst, src2, dst2, att1, False)

    h, w = _prep(out1p[0], out1p[1], bias1, Wa, ba)
    parts = _scpool(h, w.reshape(N2), batch_pad)
    return _mlp(parts[0], parts[1], W1, b1, a_prelu, W2, b2)


# speedup vs baseline: 1.0065x; 1.0065x over previous
import functools

import jax
import jax.numpy as jnp
from jax import lax
from jax.experimental import pallas as pl
from jax.experimental.pallas import tpu as pltpu
from jax.experimental.pallas import tpu_sc as plsc

N = 10000
E = 320000
D = 128
ED = 16
H = 8
O = 64
HO = 512
B = 256

NC = 2
NS = 16
NW = NC * NS
L = 16

NEG = -3.0e38

_MESH = plsc.VectorSubcoreMesh(core_axis_name="c", subcore_axis_name="s")
_SC_PARAMS = pltpu.CompilerParams(use_tc_tiling_on_sc=False)


def _mm2_body(x_ref, wl_ref, bl_ref, wr_ref, br_ref, ol_ref, or_ref):
    xb = x_ref[...]
    ol_ref[...] = jnp.dot(xb, wl_ref[...], preferred_element_type=jnp.float32) + bl_ref[...]
    or_ref[...] = jnp.dot(xb, wr_ref[...], preferred_element_type=jnp.float32) + br_ref[...]


def _node_mm0(x, Wl, bl, Wr, br):
    blk = 1000
    grid = N // blk
    return pl.pallas_call(
        _mm2_body,
        grid=(grid,),
        in_specs=[
            pl.BlockSpec((blk, D), lambda i: (i, 0)),
            pl.BlockSpec((D, HO), lambda i: (0, 0)),
            pl.BlockSpec((1, HO), lambda i: (0, 0)),
            pl.BlockSpec((D, HO), lambda i: (0, 0)),
            pl.BlockSpec((1, HO), lambda i: (0, 0)),
        ],
        out_specs=[
            pl.BlockSpec((blk, HO), lambda i: (i, 0)),
            pl.BlockSpec((blk, HO), lambda i: (i, 0)),
        ],
        out_shape=[
            jax.ShapeDtypeStruct((N, HO), jnp.float32),
            jax.ShapeDtypeStruct((N, HO), jnp.float32),
        ],
    )(x, Wl, bl.reshape(1, HO), Wr, br.reshape(1, HO))


def _mm2h_body(o0_ref, b0_ref, wl_ref, bl_ref, wr_ref, br_ref, ol_ref, or_ref):
    accl = jnp.zeros(ol_ref.shape, jnp.float32)
    accr = jnp.zeros(or_ref.shape, jnp.float32)
    for h in range(H):
        p, k = h // 2, h % 2
        hb = o0_ref[p][:, k * O:(k + 1) * O] + b0_ref[0, h]
        accl = accl + jnp.dot(hb, wl_ref[h], preferred_element_type=jnp.float32)
        accr = accr + jnp.dot(hb, wr_ref[h], preferred_element_type=jnp.float32)
    ol_ref[...] = accl + bl_ref[...]
    or_ref[...] = accr + br_ref[...]


def _node_mm1(o0, bias0, Wl, bl, Wr, br):
    blk = 1024
    grid = N2 // blk
    return pl.pallas_call(
        _mm2h_body,
        grid=(grid,),
        in_specs=[
            pl.BlockSpec((H // 2, blk, 2 * O), lambda i: (0, i, 0)),
            pl.BlockSpec((1, H, O), lambda i: (0, 0, 0)),
            pl.BlockSpec((H, O, HO), lambda i: (0, 0, 0)),
            pl.BlockSpec((1, HO), lambda i: (0, 0)),
            pl.BlockSpec((H, O, HO), lambda i: (0, 0, 0)),
            pl.BlockSpec((1, HO), lambda i: (0, 0)),
        ],
        out_specs=[
            pl.BlockSpec((blk, HO), lambda i: (i, 0)),
            pl.BlockSpec((blk, HO), lambda i: (i, 0)),
        ],
        out_shape=[
            jax.ShapeDtypeStruct((N2, HO), jnp.float32),
            jax.ShapeDtypeStruct((N2, HO), jnp.float32),
        ],
    )(o0, bias0.reshape(1, H, O), Wl.reshape(H, O, HO), bl.reshape(1, HO),
      Wr.reshape(H, O, HO), br.reshape(1, HO))


def _emm_body(ea_ref, we_ref, out_ref):
    out_ref[...] = jnp.dot(ea_ref[...], we_ref[...], preferred_element_type=jnp.float32)


def _edge_mm(edge_attr, We):
    blk = 2000
    grid = E // blk
    return pl.pallas_call(
        _emm_body,
        grid=(grid,),
        in_specs=[
            pl.BlockSpec((blk, ED), lambda i: (i, 0)),
            pl.BlockSpec((ED, HO), lambda i: (0, 0)),
        ],
        out_specs=pl.BlockSpec((blk, HO), lambda i: (i, 0)),
        out_shape=jax.ShapeDtypeStruct((E, HO), jnp.float32),
    )(edge_attr, We)


N2 = 10240
NROW2 = N2 // NS
CA = 40
EPW_A = E // NW


def _passA_body(xl_hbm, xr_hbm, e_hbm, src_hbm, dst_hbm, att_hbm,
                alpha_hbm, mx_hbm,
                sidx_v, didx_v, gl_v, gr_v, ev_v, att_v, al_v, mx_v, sem):
    c = lax.axis_index("c")
    s = lax.axis_index("s")
    wid = s * NC + c
    base0 = wid * EPW_A
    pltpu.sync_copy(att_hbm, att_v)
    att_regs = [att_v[h, pl.ds(j * L, L)] for h in range(H) for j in range(O // L)]
    iota16 = lax.iota(jnp.int32, L)
    perms = [iota16 ^ st for st in (8, 4, 2, 1)]

    def chunk_body(k, mx_carry):
        base = base0 + k * CA
        pltpu.sync_copy(src_hbm.at[pl.ds(base, CA)], sidx_v)
        pltpu.sync_copy(dst_hbm.at[pl.ds(base, CA)], didx_v)
        pltpu.sync_copy(e_hbm.at[pl.ds(base, CA)], ev_v)
        pltpu.async_copy(xl_hbm.at[sidx_v], gl_v, sem).wait()
        pltpu.async_copy(xr_hbm.at[didx_v], gr_v, sem).wait()

        def edge_body(i, mxv):
            row = jnp.zeros((L,), jnp.float32)
            for h in range(H):
                acc = jnp.zeros((L,), jnp.float32)
                for j in range(O // L):
                    sl = pl.ds(h * O + j * L, L)
                    z = gl_v[i, sl] + gr_v[i, sl] + ev_v[i, sl]
                    m = jnp.maximum(z, jnp.float32(0.2) * z)
                    acc = acc + m * att_regs[h * (O // L) + j]
                for p in perms:
                    acc = acc + jnp.take(acc, p, axis=0)
                row = jnp.where(iota16 == h, acc, row)
            al_v[i, :] = row
            return jnp.maximum(mxv, row)

        mxv2 = lax.fori_loop(0, CA, edge_body, mx_carry, unroll=False)
        pltpu.sync_copy(al_v, alpha_hbm.at[pl.ds(base, CA)])
        return mxv2

    mx_fin = lax.fori_loop(0, EPW_A // CA, chunk_body,
                           jnp.full((L,), jnp.float32(NEG)), unroll=False)
    mx_v[...] = mx_fin
    pltpu.sync_copy(mx_v, mx_hbm.at[pl.ds(wid * L, L)])


def _passA(xl, xr, e, src, dst, att):
    return pl.kernel(
        _passA_body,
        out_type=[
            jax.ShapeDtypeStruct((E, L), jnp.float32),
            jax.ShapeDtypeStruct((NW * L,), jnp.float32),
        ],
        mesh=_MESH,
        compiler_params=_SC_PARAMS,
        scratch_types=[
            pltpu.VMEM((CA,), jnp.int32),
            pltpu.VMEM((CA,), jnp.int32),
            pltpu.VMEM((CA, HO), jnp.float32),
            pltpu.VMEM((CA, HO), jnp.float32),
            pltpu.VMEM((CA, HO), jnp.float32),
            pltpu.VMEM((H, O), jnp.float32),
            pltpu.VMEM((CA, L), jnp.float32),
            pltpu.VMEM((L,), jnp.float32),
            pltpu.SemaphoreType.DMA,
        ],
    )(xl, xr, e, src, dst, att)


CB1 = 80
EPW_B1 = E // NW


def _passB1_body(alpha_hbm, mx_hbm, dst_hbm,
                 den_hbm,
                 albuf_v, didx_v, mxall_v, stage_v, den_sh, sem):
    c = lax.axis_index("c")
    s = lax.axis_index("s")
    wid = s * NC + c
    zero = jnp.zeros((L,), jnp.float32)

    def zrow(r, cc):
        stage_v[r, :] = zero
        return cc

    lax.fori_loop(0, CB1, zrow, 0, unroll=False)
    for u in range(NROW2 // CB1):
        pltpu.sync_copy(stage_v, den_sh.at[pl.ds(s * NROW2 + u * CB1, CB1)])
    plsc.subcore_barrier()

    pltpu.sync_copy(mx_hbm, mxall_v)
    mv = jnp.full((L,), jnp.float32(NEG))
    for w in range(NW):
        mv = jnp.maximum(mv, mxall_v[pl.ds(w * L, L)])

    def chunk_body(k, carry):
        base = wid * EPW_B1 + k * CB1
        pltpu.sync_copy(alpha_hbm.at[pl.ds(base, CB1)], albuf_v)
        pltpu.sync_copy(dst_hbm.at[pl.ds(base, CB1)], didx_v)

        def edge_body(r, cc):
            albuf_v[r, :] = jnp.exp(albuf_v[r, :] - mv)
            return cc

        lax.fori_loop(0, CB1, edge_body, 0, unroll=False)
        pltpu.sync_copy(albuf_v, den_sh.at[didx_v], add=True)
        return carry

    lax.fori_loop(0, EPW_B1 // CB1, chunk_body, 0, unroll=False)
    plsc.subcore_barrier()
    for u in range(NROW2 // CB1):
        pltpu.sync_copy(den_sh.at[pl.ds(s * NROW2 + u * CB1, CB1)], stage_v)
        pltpu.sync_copy(stage_v,
                        den_hbm.at[c, pl.ds(s * NROW2 + u * CB1, CB1)])


def _passB1(alpha, mx, dst):
    return pl.kernel(
        _passB1_body,
        out_type=jax.ShapeDtypeStruct((NC, N2, L), jnp.float32),
        mesh=_MESH,
        compiler_params=_SC_PARAMS,
        scratch_types=[
            pltpu.VMEM((CB1, L), jnp.float32),
            pltpu.VMEM((CB1,), jnp.int32),
            pltpu.VMEM((NW * L,), jnp.float32),
            pltpu.VMEM((CB1, L), jnp.float32),
            pltpu.VMEM_SHARED((N2, L), jnp.float32),
            pltpu.SemaphoreType.DMA,
        ],
    )(alpha, mx, dst)


def _densum_body(dp_ref, out_ref):
    out_ref[...] = dp_ref[0] + dp_ref[1]


def _densum(dp):
    blk = 2048
    return pl.pallas_call(
        _densum_body,
        grid=(N2 // blk,),
        in_specs=[pl.BlockSpec((NC, blk, L), lambda i: (0, i, 0))],
        out_specs=pl.BlockSpec((blk, L), lambda i: (i, 0)),
        out_shape=jax.ShapeDtypeStruct((N2, L), jnp.float32),
    )(dp)


CB2 = 80
EPW_B2 = E // NW


def _passB2_body(alpha_hbm, mx_hbm, den_hbm, dst_hbm,
                 a_hbm,
                 exbuf_v, denb_v, didx_v, mxall_v, sem):
    c = lax.axis_index("c")
    s = lax.axis_index("s")
    wid = s * NC + c
    eps = jnp.float32(1e-16)

    pltpu.sync_copy(mx_hbm, mxall_v)
    mv = jnp.full((L,), jnp.float32(NEG))
    for w in range(NW):
        mv = jnp.maximum(mv, mxall_v[pl.ds(w * L, L)])

    def chunk_body(k, carry):
        base = wid * EPW_B2 + k * CB2
        pltpu.sync_copy(alpha_hbm.at[pl.ds(base, CB2)], exbuf_v)
        pltpu.sync_copy(dst_hbm.at[pl.ds(base, CB2)], didx_v)
        pltpu.async_copy(den_hbm.at[didx_v], denb_v, sem).wait()

        def edge_body(r, cc):
            exbuf_v[r, :] = jnp.exp(exbuf_v[r, :] - mv) / (denb_v[r, :] + eps)
            return cc

        lax.fori_loop(0, CB2, edge_body, 0, unroll=False)
        pltpu.sync_copy(exbuf_v, a_hbm.at[pl.ds(base, CB2)])
        return carry

    lax.fori_loop(0, EPW_B2 // CB2, chunk_body, 0, unroll=False)


def _passB2(alpha, mx, den, dst):
    return pl.kernel(
        _passB2_body,
        out_type=jax.ShapeDtypeStruct((E, L), jnp.float32),
        mesh=_MESH,
        compiler_params=_SC_PARAMS,
        scratch_types=[
            pltpu.VMEM((CB2, L), jnp.float32),
            pltpu.VMEM((CB2, L), jnp.float32),
            pltpu.VMEM((CB2,), jnp.int32),
            pltpu.VMEM((NW * L,), jnp.float32),
            pltpu.SemaphoreType.DMA,
        ],
    )(alpha, mx, den, dst)


CC = 160
SW = 80
EPW_C = E // NS
HPC = H // NC
PPC = HPC // 2
W2 = 2 * O


def _passC_body(concat, a_hbm, src2_hbm, dst2_hbm, xlr_hbm,
                out_hbm,
                sidx_v, didx_v, gidx_v, a_v, g_v, acc_sh, sem):
    c = lax.axis_index("c")
    s = lax.axis_index("s")
    zero = jnp.zeros((L,), jnp.float32)
    stage_v = g_v.at[pl.ds(0, SW)]

    def zero_acc():
        def zrow(r, cc):
            for j in range(W2 // L):
                g_v[r, pl.ds(j * L, L)] = zero
            return cc

        lax.fori_loop(0, SW, zrow, 0, unroll=False)
        for u in range(NROW2 // SW):
            pltpu.sync_copy(stage_v, acc_sh.at[pl.ds(s * NROW2 + u * SW, SW)])

    def dump_acc(dst_slot):
        for u in range(NROW2 // SW):
            pltpu.sync_copy(acc_sh.at[pl.ds(s * NROW2 + u * SW, SW)], stage_v)
            pltpu.sync_copy(stage_v,
                            out_hbm.at[dst_slot, pl.ds(s * NROW2 + u * SW, SW)])

    for p_local in range(PPC):
        pair_abs = c * PPC + p_local
        h0 = 2 * pair_abs
        if concat or p_local == 0:
            zero_acc()
            plsc.subcore_barrier()

        def chunk_body(k, carry):
            base = s * EPW_C + k * CC
            pltpu.sync_copy(src2_hbm.at[pl.ds(base // SW, CC // SW)], sidx_v)
            pltpu.sync_copy(dst2_hbm.at[pl.ds(base // SW, CC // SW)], didx_v)
            pltpu.sync_copy(a_hbm.at[pl.ds(base, CC)], a_v)

            def idx_body(t, cc):
                for q in range(SW // L):
                    sl = pl.ds(q * L, L)
                    gidx_v[t, sl] = sidx_v[t, sl] * (H // 2) + pair_abs
                return cc

            lax.fori_loop(0, CC // SW, idx_body, 0, unroll=False)
            descs = [pltpu.async_copy(xlr_hbm.at[gidx_v.at[u]],
                                      g_v.at[pl.ds(u * SW, SW)], sem)
                     for u in range(CC // SW)]
            for d in descs:
                d.wait()

            def edge_body(r, cc):
                av0 = jnp.take(a_v[r, :], jnp.full((L,), h0, jnp.int32),
                               axis=0)
                av1 = jnp.take(a_v[r, :], jnp.full((L,), h0 + 1, jnp.int32),
                               axis=0)
                for j in range(O // L):
                    sl = pl.ds(j * L, L)
                    g_v[r, sl] = g_v[r, sl] * av0
                for j in range(O // L):
                    sl = pl.ds(O + j * L, L)
                    g_v[r, sl] = g_v[r, sl] * av1
                return cc

            lax.fori_loop(0, CC, edge_body, 0, unroll=False)
            for u in range(CC // SW):
                pltpu.sync_copy(g_v.at[pl.ds(u * SW, SW)],
                                acc_sh.at[didx_v.at[u]], add=True)
            return carry

        lax.fori_loop(0, EPW_C // CC, chunk_body, 0, unroll=False)
        plsc.subcore_barrier()
        if concat:
            dump_acc(pair_abs)
            plsc.subcore_barrier()
    if not concat:
        dump_acc(c)


def _passC(a, src2, dst2, xlr, concat):
    slots = H // 2 if concat else NC
    return pl.kernel(
        functools.partial(_passC_body, concat),
        out_type=jax.ShapeDtypeStruct((slots, N2, W2), jnp.float32),
        mesh=_MESH,
        compiler_params=_SC_PARAMS,
        scratch_types=[
            pltpu.VMEM((CC // SW, SW), jnp.int32),
            pltpu.VMEM((CC // SW, SW), jnp.int32),
            pltpu.VMEM((CC // SW, SW), jnp.int32),
            pltpu.VMEM((CC, L), jnp.float32),
            pltpu.VMEM((CC, W2), jnp.float32),
            pltpu.VMEM_SHARED((N2, W2), jnp.float32),
            pltpu.SemaphoreType.DMA,
        ],
    )(a, src2, dst2, xlr)


PPB = 1024


def _prep_body(p0_ref, p1_ref, bias1_ref, wa_ref, ba_ref, h_ref, w_ref):
    ps = p0_ref[...] + p1_ref[...]
    h = (ps[:, :O] + ps[:, O:]) * jnp.float32(1.0 / H) + bias1_ref[...]
    z = jnp.dot(h, wa_ref[...], preferred_element_type=jnp.float32) + ba_ref[...]
    h_ref[...] = h
    w_ref[...] = jnp.float32(1.0) / (jnp.float32(1.0) + jnp.exp(-z))


def _prep(p0, p1, bias1, Wa, ba):
    grid = N2 // PPB
    return pl.pallas_call(
        _prep_body,
        grid=(grid,),
        in_specs=[
            pl.BlockSpec((PPB, W2), lambda i: (i, 0)),
            pl.BlockSpec((PPB, W2), lambda i: (i, 0)),
            pl.BlockSpec((1, O), lambda i: (0, 0)),
            pl.BlockSpec((O, 1), lambda i: (0, 0)),
            pl.BlockSpec((1, 1), lambda i: (0, 0)),
        ],
        out_specs=[
            pl.BlockSpec((PPB, O), lambda i: (i, 0)),
            pl.BlockSpec((PPB, 1), lambda i: (i, 0)),
        ],
        out_shape=[
            jax.ShapeDtypeStruct((N2, O), jnp.float32),
            jax.ShapeDtypeStruct((N2, 1), jnp.float32),
        ],
    )(p0, p1, bias1.reshape(1, O), Wa, ba.reshape(1, 1))


RPW = N2 // NW
BT = 264


def _scpool_body(h_hbm, w_hbm, batch_hbm, out_hbm,
                 h_v, w_v, b_v, tab1_v, tab2_v, sem):
    c = lax.axis_index("c")
    s = lax.axis_index("s")
    wid = s * NC + c
    rows0 = wid * RPW
    pltpu.sync_copy(h_hbm.at[pl.ds(rows0, RPW)], h_v)
    pltpu.sync_copy(w_hbm.at[pl.ds(rows0, RPW)], w_v)
    pltpu.sync_copy(batch_hbm.at[pl.ds(rows0, RPW)], b_v)
    zero = jnp.zeros((L,), jnp.float32)
    neg = jnp.full((L,), jnp.float32(NEG))

    def zrow(r, cc):
        for j in range(O // L):
            tab1_v[r, pl.ds(j * L, L)] = neg
            tab2_v[r, pl.ds(j * L, L)] = zero
        return cc

    lax.fori_loop(0, BT, zrow, 0, unroll=False)

    def grp(g, cc):
        seg16 = b_v[pl.ds(g * L, L)]
        w16 = w_v[pl.ds(g * L, L)]
        for t in range(L):
            r = g * L + t
            seg = seg16[t]
            wb = jnp.full((L,), w16[t])
            for j in range(O // L):
                sl = pl.ds(j * L, L)
                hv = h_v[r, sl]
                tab1_v[seg, sl] = jnp.maximum(tab1_v[seg, sl], hv)
                tab2_v[seg, sl] = tab2_v[seg, sl] + hv * wb
        return cc

    lax.fori_loop(0, RPW // L, grp, 0, unroll=False)
    pltpu.sync_copy(tab1_v.at[pl.ds(0, B)], out_hbm.at[0, wid])
    pltpu.sync_copy(tab2_v.at[pl.ds(0, B)], out_hbm.at[1, wid])


def _scpool(h, w, batch_pad):
    return pl.kernel(
        _scpool_body,
        out_type=jax.ShapeDtypeStruct((2, NW, B, O), jnp.float32),
        mesh=_MESH,
        compiler_params=_SC_PARAMS,
        scratch_types=[
            pltpu.VMEM((RPW, O), jnp.float32),
            pltpu.VMEM((RPW,), jnp.float32),
            pltpu.VMEM((RPW,), jnp.int32),
            pltpu.VMEM((BT, O), jnp.float32),
            pltpu.VMEM((BT, O), jnp.float32),
            pltpu.SemaphoreType.DMA,
        ],
    )(h, w, batch_pad)


def _mlp_body(g1p_ref, g2p_ref, w1_ref, b1_ref, ap_ref, w2_ref, b2_ref, out_ref):
    g1 = jnp.max(g1p_ref[...], axis=0)
    g1 = jnp.where(g1 <= jnp.float32(-1e38), jnp.float32(0.0), g1)
    g2 = jnp.sum(g2p_ref[...], axis=0)
    g = jnp.concatenate([g1, g2], axis=1)
    gm = jnp.dot(g, w1_ref[...], preferred_element_type=jnp.float32) + b1_ref[...]
    gm = jnp.where(gm >= 0, gm, ap_ref[0, 0] * gm)
    out_ref[...] = jnp.dot(gm, w2_ref[...], preferred_element_type=jnp.float32) + b2_ref[...]


def _mlp(g1p, g2p, W1, b1, a_prelu, W2, b2):
    return pl.pallas_call(
        _mlp_body,
        out_shape=jax.ShapeDtypeStruct((B, 1), jnp.float32),
    )(g1p, g2p, W1, b1.reshape(1, -1), a_prelu.reshape(1, 1), W2, b2.reshape(1, 1))


def _gat_layer_sc(xl, xr, e, src, dst, src2, dst2, att, concat):
    alpha, mx = _passA(xl, xr, e, src, dst, att)
    denp = _passB1(alpha, mx, dst)
    a = _passB2(alpha, mx, _densum(denp), dst)
    nn = xl.shape[0]
    return _passC(a, src2, dst2, xl.reshape(nn * (H // 2), W2), concat)


def kernel(x, edge_index, edge_attr, batch, Wl0, bl0, Wr0, br0, We0, att0,
           bias0, Wl1, bl1, Wr1, br1, We1, att1, bias1, Wa, ba, W1, b1,
           a_prelu, W2, b2):
    src = edge_index[0]
    dst = edge_index[1]
    src2 = src.reshape(E // 80, 80)
    dst2 = dst.reshape(E // 80, 80)
    batch_pad = jnp.concatenate(
        [batch, jnp.full((N2 - N,), B, jnp.int32)])

    xl0, xr0 = _node_mm0(x, Wl0, bl0, Wr0, br0)
    e0 = _edge_mm(edge_attr, We0)
    out0 = _gat_layer_sc(xl0, xr0, e0, src, dst, src2, dst2, att0, True)

    xl1, xr1 = _node_mm1(out0, bias0, Wl1, bl1, Wr1, br1)
    e1 = _edge_mm(edge_attr, We1)
    out1p = _gat_layer_sc(xl1, xr1, e1, src, dst, src2, dst2, att1, False)

    h, w = _prep(out1p[0], out1p[1], bias1, Wa, ba)
    parts = _scpool(h, w.reshape(N2), batch_pad)
    return _mlp(parts[0], parts[1], W1, b1, a_prelu, W2, b2)
